# Initial kernel scaffold; baseline (speedup 1.0000x reference)
import jax
import jax.numpy as jnp
from jax.experimental import pallas as pl


def kernel(A_batch, feature, no_sc_idx, no_fc_idx, W_sc0, b_sc0, W_sc1, b_sc1, W_sc2, b_sc2, W_fc0, b_fc0, W_fc1, b_fc1, W_fc2, b_fc2, w1_w, w1_b, w2_w, w2_b, attention, out_w, out_b):
    raise NotImplementedError("write your pallas kernel here")


# speedup vs baseline: 1.7853x; 1.7853x over previous
import jax
import jax.numpy as jnp
from jax.experimental import pallas as pl
from jax.experimental.pallas import tpu as pltpu

_B, _N, _F, _H, _OUT, _K = 32, 256, 512, 512, 8, 5


def _dot(a, b):
    return jax.lax.dot(a, b, preferred_element_type=jnp.float32)


def _gcn_body(A_ref, x_ref,
              Ws0, bs0, Ws1, bs1, Ws2, bs2,
              Wf0, bf0, Wf1, bf1, Wf2, bf2,
              w1w_ref, w2w_ref, attn_ref,
              es_ref, ef_ref, gs_ref, gf_ref, s1_ref, s2_ref):
    A_s = A_ref[0, 0]
    A_f = A_ref[0, 1]
    x = x_ref[0]
    xs = x
    acc_s = jnp.zeros((_N, _H), jnp.float32)
    for (W, b) in ((Ws0, bs0), (Ws1, bs1), (Ws2, bs2)):
        xs = jax.nn.relu(_dot(A_s, _dot(xs, W[...])) + b[...])
        acc_s = acc_s + xs
    xf = x
    acc_f = jnp.zeros((_N, _H), jnp.float32)
    for (W, b) in ((Wf0, bf0), (Wf1, bf1), (Wf2, bf2)):
        xf = jax.nn.relu(_dot(A_f, _dot(xf, W[...])) + b[...])
        acc_f = acc_f + xf
    E_s = acc_s * (1.0 / 3.0)
    E_f = acc_f * (1.0 / 3.0)
    es_ref[0] = E_s
    ef_ref[0] = E_f
    gs_ref[0] = jnp.mean(E_s, axis=0, keepdims=True)
    gf_ref[0] = jnp.mean(E_f, axis=0, keepdims=True)
    dn = (((1,), (1,)), ((), ()))
    v1row = jax.lax.dot_general(attn_ref[...], w1w_ref[...], dn,
                                preferred_element_type=jnp.float32)
    v2row = jax.lax.dot_general(attn_ref[...], w2w_ref[...], dn,
                                preferred_element_type=jnp.float32)
    s1_ref[0] = jax.lax.dot_general(v1row, E_s, dn,
                                    preferred_element_type=jnp.float32)
    s2_ref[0] = jax.lax.dot_general(v2row, E_f, dn,
                                    preferred_element_type=jnp.float32)


def _knn_body(gs_ref, gf_ref, s1_ref, s2_ref,
              msc_row_ref, msc_col_ref, mfc_row_ref, mfc_col_ref,
              w1b_ref, w2b_ref, attn_ref,
              a1_ref, a2_ref, idx1_ref, idx2_ref):
    r_iota = jax.lax.broadcasted_iota(jnp.int32, (_B, _B), 0)
    c_iota = jax.lax.broadcasted_iota(jnp.int32, (_B, _B), 1)
    eye_b = r_iota == c_iota
    ones_row = jnp.ones((1, _H), jnp.float32)
    dn = (((1,), (1,)), ((), ()))

    def topk_combine(g, mask_row, mask_col):
        diff = g[:, None, :] - g[None, :, :]
        d = jnp.sum(diff * diff, axis=-1)
        bad = eye_b | (mask_row[...] > 0.5)
        d = jnp.where(bad, jnp.inf, d)
        comb = jnp.zeros((_B, _B), jnp.float32)
        taken = jnp.zeros((_B, _B), jnp.bool_)
        idx_cols = []
        for _ in range(_K):
            d_eff = jnp.where(taken, jnp.inf, d)
            m = jnp.min(d_eff, axis=1, keepdims=True)
            cand = jnp.where((d_eff <= m) & (~taken), c_iota, _B)
            amin = jnp.min(cand, axis=1, keepdims=True)
            onehot = c_iota == amin
            comb = comb + onehot.astype(jnp.float32) * (1.0 / _K)
            idx_cols.append(amin)
            taken = taken | onehot
        null_col = mask_col[...] > 0.5
        comb = jnp.where(null_col, comb, eye_b.astype(jnp.float32))
        idx = jnp.concatenate(idx_cols, axis=1)
        r_bk = jax.lax.broadcasted_iota(jnp.int32, (_B, _K), 0)
        idx = jnp.where(null_col, idx, r_bk)
        return comb, idx

    comb1, idx1 = topk_combine(gf_ref[...], msc_row_ref, msc_col_ref)
    comb2, idx2 = topk_combine(gs_ref[...], mfc_row_ref, mfc_col_ref)
    idx1_ref[...] = idx1
    idx2_ref[...] = idx2

    c1 = jnp.sum(w1b_ref[...] * attn_ref[...])
    c2 = jnp.sum(w2b_ref[...] * attn_ref[...])
    z1 = _dot(comb1, s1_ref[...]) + c1
    z2 = _dot(comb2, s2_ref[...]) + c2
    g1 = jnp.where(z1 >= 0, z1, 0.3 * z1)
    g2 = jnp.where(z2 >= 0, z2, 0.3 * z2)
    m = jnp.maximum(g1, g2)
    e1 = jnp.exp(g1 - m)
    e2 = jnp.exp(g2 - m)
    denom = e1 + e2
    a1_ref[...] = e1 / denom
    a2_ref[...] = e2 / denom


def _fuse_body(i1_ref, i2_ref, *refs):
    es_refs = refs[:_K]
    ef_refs = refs[_K:2 * _K]
    a1_ref, a2_ref, ow_ref, ob_ref, out_ref = refs[2 * _K:]
    b = pl.program_id(0)
    Eb1 = es_refs[0][0]
    for r in es_refs[1:]:
        Eb1 = Eb1 + r[0]
    Eb2 = ef_refs[0][0]
    for r in ef_refs[1:]:
        Eb2 = Eb2 + r[0]
    a1row = a1_ref[pl.ds(b, 1), :]
    a2row = a2_ref[pl.ds(b, 1), :]
    pool = _dot(a1row, Eb1) + _dot(a2row, Eb2)
    pool = pool * (1.0 / (_N * _K))
    out_ref[pl.ds(b, 1), :] = _dot(pool, ow_ref[...]) + ob_ref[...]


def kernel(A_batch, feature, no_sc_idx, no_fc_idx,
           W_sc0, b_sc0, W_sc1, b_sc1, W_sc2, b_sc2,
           W_fc0, b_fc0, W_fc1, b_fc1, W_fc2, b_fc2,
           w1_w, w1_b, w2_w, w2_b, attention, out_w, out_b):
    f32 = jnp.float32
    attn_row = attention.reshape(1, _H)
    msc = no_sc_idx.astype(f32)
    mfc = no_fc_idx.astype(f32)

    const = lambda shape: pl.BlockSpec(shape, lambda b: tuple(0 for _ in shape))
    k1 = pl.pallas_call(
        _gcn_body,
        grid=(_B,),
        in_specs=[
            pl.BlockSpec((1, 2, _N, _N), lambda b: (b, 0, 0, 0)),
            pl.BlockSpec((1, _N, _F), lambda b: (b, 0, 0)),
            const((_F, _H)), const((1, _H)),
            const((_H, _H)), const((1, _H)),
            const((_H, _H)), const((1, _H)),
            const((_F, _H)), const((1, _H)),
            const((_H, _H)), const((1, _H)),
            const((_H, _H)), const((1, _H)),
            const((_H, _H)), const((_H, _H)), const((1, _H)),
        ],
        out_specs=[
            pl.BlockSpec((1, _N, _H), lambda b: (b, 0, 0)),
            pl.BlockSpec((1, _N, _H), lambda b: (b, 0, 0)),
            pl.BlockSpec((1, 1, _H), lambda b: (b, 0, 0)),
            pl.BlockSpec((1, 1, _H), lambda b: (b, 0, 0)),
            pl.BlockSpec((1, 1, _N), lambda b: (b, 0, 0)),
            pl.BlockSpec((1, 1, _N), lambda b: (b, 0, 0)),
        ],
        out_shape=[
            jax.ShapeDtypeStruct((_B, _N, _H), f32),
            jax.ShapeDtypeStruct((_B, _N, _H), f32),
            jax.ShapeDtypeStruct((_B, 1, _H), f32),
            jax.ShapeDtypeStruct((_B, 1, _H), f32),
            jax.ShapeDtypeStruct((_B, 1, _N), f32),
            jax.ShapeDtypeStruct((_B, 1, _N), f32),
        ],
    )
    es, ef, gs, gf, s1, s2 = k1(
        A_batch, feature,
        W_sc0, b_sc0.reshape(1, _H), W_sc1, b_sc1.reshape(1, _H),
        W_sc2, b_sc2.reshape(1, _H),
        W_fc0, b_fc0.reshape(1, _H), W_fc1, b_fc1.reshape(1, _H),
        W_fc2, b_fc2.reshape(1, _H),
        w1_w, w2_w, attn_row)

    k2 = pl.pallas_call(
        _knn_body,
        out_shape=[
            jax.ShapeDtypeStruct((_B, _N), f32),
            jax.ShapeDtypeStruct((_B, _N), f32),
            jax.ShapeDtypeStruct((_B, _K), jnp.int32),
            jax.ShapeDtypeStruct((_B, _K), jnp.int32),
        ],
    )
    a1, a2, idx1, idx2 = k2(
        gs.reshape(_B, _H), gf.reshape(_B, _H),
        s1.reshape(_B, _N), s2.reshape(_B, _N),
        msc.reshape(1, _B), msc.reshape(_B, 1),
        mfc.reshape(1, _B), mfc.reshape(_B, 1),
        w1_b.reshape(1, _H), w2_b.reshape(1, _H), attn_row)

    def gat1(k):
        return pl.BlockSpec((1, _N, _H), lambda b, i1, i2, k=k: (i1[b, k], 0, 0))

    def gat2(k):
        return pl.BlockSpec((1, _N, _H), lambda b, i1, i2, k=k: (i2[b, k], 0, 0))

    k3 = pl.pallas_call(
        _fuse_body,
        grid_spec=pltpu.PrefetchScalarGridSpec(
            num_scalar_prefetch=2,
            grid=(_B,),
            in_specs=[
                *[gat1(k) for k in range(_K)],
                *[gat2(k) for k in range(_K)],
                pl.BlockSpec((_B, _N), lambda b, i1, i2: (0, 0)),
                pl.BlockSpec((_B, _N), lambda b, i1, i2: (0, 0)),
                pl.BlockSpec((_H, _OUT), lambda b, i1, i2: (0, 0)),
                pl.BlockSpec((1, _OUT), lambda b, i1, i2: (0, 0)),
            ],
            out_specs=pl.BlockSpec((_B, _OUT), lambda b, i1, i2: (0, 0)),
        ),
        out_shape=jax.ShapeDtypeStruct((_B, _OUT), f32),
    )
    return k3(idx1, idx2,
              es, es, es, es, es,
              ef, ef, ef, ef, ef,
              a1, a2, out_w, out_b.reshape(1, _OUT))
